# baseline (speedup 1.0000x reference)
import dataclasses

import jax
import jax.numpy as jnp
from jax import lax
from jax.experimental import pallas as pl
from jax.experimental.pallas import tpu as pltpu
from jax.experimental.pallas import tpu_sc as plsc

L = 16
C = 2048
U = 4

_CES = ((0, 1), (3, 4), (2, 4), (2, 3), (1, 4), (0, 3), (0, 2), (1, 3), (1, 2))


def _group(x_vmem, sorted_vmem, hist_vmem, sl):
    d = [x_vmem[j, sl] for j in range(5)]

    s = None
    for j in range(5):
        e = d[j].astype(jnp.int32)
        t = jnp.int32(1) << (3 * e)
        s = t if s is None else s + t
    for v in range(6):
        h = (s >> (3 * v + 3)) & 7 if v < 5 else s >> 18
        hist_vmem[v, sl] = h.astype(jnp.float32)

    c = list(d)
    for a, b in _CES:
        lo = jnp.minimum(c[a], c[b])
        hi = jnp.maximum(c[a], c[b])
        c[a], c[b] = lo, hi
    for j in range(5):
        sorted_vmem[j, sl] = c[j]


def _block_body(x_vmem, sorted_vmem, hist_vmem):
    @pl.loop(0, C, step=L * U)
    def _(c0):
        for u in range(U):
            _group(x_vmem, sorted_vmem, hist_vmem, pl.ds(c0 + u * L, L))


def kernel(dice_state):
    B = dice_state.shape[0]
    x_t = dice_state.T
    mesh = plsc.VectorSubcoreMesh(core_axis_name="c", subcore_axis_name="s")
    cp = pltpu.CompilerParams()
    fields = pltpu.CompilerParams.__dataclass_fields__
    if "needs_layout_passes" in fields:
        cp = dataclasses.replace(cp, needs_layout_passes=False)
    if "use_tc_tiling_on_sc" in fields:
        cp = dataclasses.replace(cp, use_tc_tiling_on_sc=False)

    @pl.kernel(
        out_type=(
            jax.ShapeDtypeStruct((5, B), jnp.float32),
            jax.ShapeDtypeStruct((6, B), jnp.float32),
        ),
        mesh=mesh,
        compiler_params=cp,
    )
    def run(x_hbm, sorted_hbm, hist_hbm):
        pltpu.emit_pipeline(
            _block_body,
            grid=(B // C,),
            in_specs=[pl.BlockSpec((5, C), lambda i: (0, i))],
            out_specs=[
                pl.BlockSpec((5, C), lambda i: (0, i)),
                pl.BlockSpec((6, C), lambda i: (0, i)),
            ],
            core_axis_name=("c", "s"),
            dimension_semantics=(pltpu.PARALLEL,),
        )(x_hbm, sorted_hbm, hist_hbm)

    sorted_t, hist_t = run(x_t)
    return sorted_t.T, hist_t.T


# speedup vs baseline: 22.2958x; 22.2958x over previous
import dataclasses

import jax
import jax.numpy as jnp
from jax import lax
from jax.experimental import pallas as pl
from jax.experimental.pallas import tpu as pltpu
from jax.experimental.pallas import tpu_sc as plsc

L = 16
C = 2048
U = 4

_CES = ((0, 1), (3, 4), (2, 4), (2, 3), (1, 4), (0, 3), (0, 2), (1, 3), (1, 2))


def _group(x_vmem, sorted_vmem, hist_vmem, sl):
    d = [x_vmem[j, sl] for j in range(5)]

    s = None
    for j in range(5):
        e = d[j].astype(jnp.int32)
        t = jnp.int32(1) << (3 * e)
        s = t if s is None else s + t
    for v in range(6):
        h = (s >> (3 * v + 3)) & 7 if v < 5 else s >> 18
        hist_vmem[v, sl] = h.astype(jnp.float32)

    c = list(d)
    for a, b in _CES:
        lo = jnp.minimum(c[a], c[b])
        hi = jnp.maximum(c[a], c[b])
        c[a], c[b] = lo, hi
    for j in range(5):
        sorted_vmem[j, sl] = c[j]


def _block_body(x_vmem, sorted_vmem, hist_vmem):
    @pl.loop(0, C, step=L * U)
    def _(c0):
        for u in range(U):
            _group(x_vmem, sorted_vmem, hist_vmem, pl.ds(c0 + u * L, L))


def kernel(dice_state):
    B = dice_state.shape[0]
    x_t = dice_state.T
    mesh = plsc.VectorSubcoreMesh(core_axis_name="c", subcore_axis_name="s")
    cp = pltpu.CompilerParams()
    fields = pltpu.CompilerParams.__dataclass_fields__
    if "needs_layout_passes" in fields:
        cp = dataclasses.replace(cp, needs_layout_passes=False)
    if "use_tc_tiling_on_sc" in fields:
        cp = dataclasses.replace(cp, use_tc_tiling_on_sc=True)

    @pl.kernel(
        out_type=(
            jax.ShapeDtypeStruct((5, B), jnp.float32),
            jax.ShapeDtypeStruct((6, B), jnp.float32),
        ),
        mesh=mesh,
        compiler_params=cp,
    )
    def run(x_hbm, sorted_hbm, hist_hbm):
        pltpu.emit_pipeline(
            _block_body,
            grid=(B // C,),
            in_specs=[pl.BlockSpec((5, C), lambda i: (0, i))],
            out_specs=[
                pl.BlockSpec((5, C), lambda i: (0, i)),
                pl.BlockSpec((6, C), lambda i: (0, i)),
            ],
            core_axis_name=("c", "s"),
            dimension_semantics=(pltpu.PARALLEL,),
        )(x_hbm, sorted_hbm, hist_hbm)

    sorted_t, hist_t = run(x_t)
    return sorted_t.T, hist_t.T


# speedup vs baseline: 24.4808x; 1.0980x over previous
import dataclasses

import jax
import jax.numpy as jnp
from jax import lax
from jax.experimental import pallas as pl
from jax.experimental.pallas import tpu as pltpu
from jax.experimental.pallas import tpu_sc as plsc

L = 16
C = 2048
U = 4

_CES = ((0, 1), (3, 4), (2, 4), (2, 3), (1, 4), (0, 3), (0, 2), (1, 3), (1, 2))


def _group(x_vmem, sorted_vmem, hist_vmem, sl):
    d = [x_vmem[j, sl] for j in range(5)]

    s = None
    for j in range(5):
        e = d[j].astype(jnp.int32)
        t = jnp.int32(1) << (3 * e)
        s = t if s is None else s + t
    for v in range(6):
        h = (s >> (3 * v + 3)) & 7 if v < 5 else s >> 18
        hist_vmem[v, sl] = h.astype(jnp.float32)

    c = list(d)
    for a, b in _CES:
        lo = jnp.minimum(c[a], c[b])
        hi = jnp.maximum(c[a], c[b])
        c[a], c[b] = lo, hi
    for j in range(5):
        sorted_vmem[j, sl] = c[j]


def _block_body(x_vmem, sorted_vmem, hist_vmem):
    @plsc.parallel_loop(0, C, step=L, unroll=U)
    def _(c0):
        _group(x_vmem, sorted_vmem, hist_vmem, pl.ds(c0, L))


def kernel(dice_state):
    B = dice_state.shape[0]
    x_t = dice_state.T
    mesh = plsc.VectorSubcoreMesh(core_axis_name="c", subcore_axis_name="s")
    cp = pltpu.CompilerParams()
    fields = pltpu.CompilerParams.__dataclass_fields__
    if "needs_layout_passes" in fields:
        cp = dataclasses.replace(cp, needs_layout_passes=False)
    if "use_tc_tiling_on_sc" in fields:
        cp = dataclasses.replace(cp, use_tc_tiling_on_sc=True)

    @pl.kernel(
        out_type=(
            jax.ShapeDtypeStruct((5, B), jnp.float32),
            jax.ShapeDtypeStruct((6, B), jnp.float32),
        ),
        mesh=mesh,
        compiler_params=cp,
    )
    def run(x_hbm, sorted_hbm, hist_hbm):
        pltpu.emit_pipeline(
            _block_body,
            grid=(B // C,),
            in_specs=[pl.BlockSpec((5, C), lambda i: (0, i))],
            out_specs=[
                pl.BlockSpec((5, C), lambda i: (0, i)),
                pl.BlockSpec((6, C), lambda i: (0, i)),
            ],
            core_axis_name=("c", "s"),
            dimension_semantics=(pltpu.PARALLEL,),
        )(x_hbm, sorted_hbm, hist_hbm)

    sorted_t, hist_t = run(x_t)
    return sorted_t.T, hist_t.T
